# baseline (speedup 1.0000x reference)
import functools

import jax
import jax.numpy as jnp
from jax import lax
from jax.experimental import pallas as pl
from jax.experimental.pallas import tpu as pltpu
from jax.experimental.pallas import tpu_sc as plsc

E = 64
D = 768
F = 3072
T = 4096
C = 80
CP = 88
SLOTS = E * CP
TPAD = T + 8
ZERO_SLOT = C

NC, NS = 2, 16
NW = NC * NS

BLK = 512
NB = T // BLK

FB = 768
NFB = F // FB


def _router_body(x_ref, wr_ref, fidx_ref, cidx_ref, gate_ref, aux_ref):
    x = x_ref[...]
    wr = wr_ref[...]
    logits = lax.dot_general(
        x.astype(jnp.bfloat16), wr.astype(jnp.bfloat16),
        (((1,), (0,)), ((), ())),
        preferred_element_type=jnp.float32)
    m = jnp.max(logits, axis=-1, keepdims=True)
    el = jnp.exp(logits - m)
    s = jnp.sum(el, axis=-1, keepdims=True)
    probs = el / s
    maxp = jnp.max(probs, axis=-1, keepdims=True)
    iota_e = lax.broadcasted_iota(jnp.int32, (T, E), 1)
    cand = jnp.where(probs >= maxp, iota_e, E)
    eidx = jnp.min(cand, axis=-1, keepdims=True)
    onehot = (iota_e == eidx).astype(jnp.float32)

    ti = lax.broadcasted_iota(jnp.int32, (BLK, BLK), 0)
    tj = lax.broadcasted_iota(jnp.int32, (BLK, BLK), 1)
    tri = (tj <= ti).astype(jnp.float32)
    carry = jnp.zeros((1, E), jnp.float32)
    for r in range(NB):
        blk = onehot[r * BLK:(r + 1) * BLK, :]
        within = lax.dot_general(
            tri, blk, (((1,), (0,)), ((), ())),
            preferred_element_type=jnp.float32)
        tot = within + carry
        pos = jnp.sum(tot * blk, axis=-1, keepdims=True) - 1.0
        posi = pos.astype(jnp.int32)
        eb = eidx[r * BLK:(r + 1) * BLK, :]
        keep = posi < C
        slot = eb * CP + posi
        fidx_ref[r * BLK:(r + 1) * BLK, :] = jnp.where(keep, slot, SLOTS)
        cidx_ref[r * BLK:(r + 1) * BLK, :] = jnp.where(keep, slot, ZERO_SLOT)
        carry = carry + within[BLK - 1:BLK, :]
    gate_ref[...] = maxp
    pmean = jnp.sum(probs, axis=0, keepdims=True) * (1.0 / T)
    aux_ref[...] = (E / T) * jnp.sum(carry * pmean, axis=1, keepdims=True)


@jax.jit
def _router(x, wr):
    return pl.pallas_call(
        _router_body,
        out_shape=[
            jax.ShapeDtypeStruct((T, 1), jnp.int32),
            jax.ShapeDtypeStruct((T, 1), jnp.int32),
            jax.ShapeDtypeStruct((T, 1), jnp.float32),
            jax.ShapeDtypeStruct((1, 1), jnp.float32),
        ],
    )(x, wr)


def _dispatch_body(x_hbm, fidx_hbm, gate_hbm, buf_hbm, gslot_hbm,
                   fidx_v, stt_v, gate_v, gsf_v, idx_v, rows_v, sem):
    wid = lax.axis_index("c") * NS + lax.axis_index("s")
    pltpu.sync_copy(fidx_hbm, fidx_v)
    pltpu.sync_copy(gate_hbm, gate_v)

    @plsc.parallel_loop(0, SLOTS // 16, unroll=8)
    def _(i):
        stt_v[pl.ds(i * 16, 16)] = jnp.zeros((16,), jnp.int32)
        gsf_v[pl.ds(i * 16, 16)] = jnp.zeros((16,), jnp.float32)

    @plsc.parallel_loop(0, T // 16, unroll=8)
    def _(i):
        idx = fidx_v[pl.ds(i * 16, 16)]
        vals = lax.iota(jnp.int32, 16) + i * 16
        msk = idx < SLOTS
        plsc.store_scatter(stt_v, [idx], vals, mask=msk)
        plsc.store_scatter(gsf_v, [idx], gate_v[pl.ds(i * 16, 16)], mask=msk)

    base = wid * (SLOTS // NW)
    for j in range(2):
        cb = base + j * CP
        for k in (0, 16, 32, 48, 64, 72):
            idx_v[pl.ds(k, 16)] = stt_v[pl.ds(cb + k, 16)]
        pltpu.async_copy(x_hbm.at[idx_v], rows_v, sem).wait()
        pltpu.sync_copy(rows_v, buf_hbm.at[pl.ds(cb, CP)])
    pltpu.sync_copy(gsf_v.at[pl.ds(base, SLOTS // NW)],
                    gslot_hbm.at[pl.ds(base, SLOTS // NW)])


@jax.jit
def _dispatch(x, fidx, gate):
    return pl.kernel(
        _dispatch_body,
        out_type=[
            jax.ShapeDtypeStruct((SLOTS, D), jnp.float32),
            jax.ShapeDtypeStruct((SLOTS,), jnp.float32),
        ],
        mesh=plsc.VectorSubcoreMesh(
            core_axis_name="c", subcore_axis_name="s",
            num_cores=NC, num_subcores=NS),
        compiler_params=pltpu.CompilerParams(needs_layout_passes=False),
        scratch_types=[
            pltpu.VMEM((T,), jnp.int32),
            pltpu.VMEM((SLOTS,), jnp.int32),
            pltpu.VMEM((T,), jnp.float32),
            pltpu.VMEM((SLOTS,), jnp.float32),
            pltpu.VMEM((CP,), jnp.int32),
            pltpu.VMEM((CP, D), jnp.float32),
            pltpu.SemaphoreType.DMA,
        ],
    )(x, fidx, gate)


def _ffn_body(buf_ref, w1_ref, w2_ref, gs_ref, y_ref):
    f = pl.program_id(1)
    xb = buf_ref[0].astype(jnp.bfloat16)
    w1 = w1_ref[0].astype(jnp.bfloat16)
    h = lax.dot_general(xb, w1, (((1,), (0,)), ((), ())),
                        preferred_element_type=jnp.float32)
    h = jnp.maximum(h, 0.0).astype(jnp.bfloat16)
    w2 = w2_ref[0].astype(jnp.bfloat16)
    yp = lax.dot_general(h, w2, (((1,), (0,)), ((), ())),
                         preferred_element_type=jnp.float32)

    @pl.when(f == 0)
    def _():
        y_ref[...] = jnp.zeros_like(y_ref)

    y_ref[0] += yp

    @pl.when(f == NFB - 1)
    def _():
        y_ref[0] = y_ref[0] * gs_ref[0]


@jax.jit
def _ffn(buf, w1, w2, gslot):
    return pl.pallas_call(
        _ffn_body,
        grid=(E, NFB),
        in_specs=[
            pl.BlockSpec((1, CP, D), lambda e, f: (e, 0, 0)),
            pl.BlockSpec((1, D, FB), lambda e, f: (e, 0, f)),
            pl.BlockSpec((1, FB, D), lambda e, f: (e, f, 0)),
            pl.BlockSpec((1, CP, 1), lambda e, f: (e, 0, 0)),
        ],
        out_specs=pl.BlockSpec((1, CP, D), lambda e, f: (e, 0, 0)),
        out_shape=jax.ShapeDtypeStruct((E, CP, D), jnp.float32),
    )(buf, w1, w2, gslot)


def _combine_body(y_hbm, cidx_hbm, out_hbm, cidx_v, rows_v, sem):
    wid = lax.axis_index("c") * NS + lax.axis_index("s")
    base = wid * (T // NW)
    pltpu.sync_copy(cidx_hbm.at[pl.ds(base, T // NW)], cidx_v)
    pltpu.async_copy(y_hbm.at[cidx_v], rows_v, sem).wait()
    pltpu.sync_copy(rows_v, out_hbm.at[pl.ds(base, T // NW)])


@jax.jit
def _combine(y_flat, cidx):
    return pl.kernel(
        _combine_body,
        out_type=jax.ShapeDtypeStruct((T, D), jnp.float32),
        mesh=plsc.VectorSubcoreMesh(
            core_axis_name="c", subcore_axis_name="s",
            num_cores=NC, num_subcores=NS),
        compiler_params=pltpu.CompilerParams(needs_layout_passes=False),
        scratch_types=[
            pltpu.VMEM((T // NW,), jnp.int32),
            pltpu.VMEM((T // NW, D), jnp.float32),
            pltpu.SemaphoreType.DMA,
        ],
    )(y_flat, cidx)


def kernel(x, Wr, W1, W2):
    fidx, cidx, gate, aux = _router(x, Wr)
    buf, gslot = _dispatch(x, fidx.reshape(T), gate.reshape(T))
    y = _ffn(buf.reshape(E, CP, D), W1, W2, gslot.reshape(E, CP, 1))
    out = _combine(y.reshape(SLOTS, D), cidx.reshape(T))
    return out, aux.reshape(())


# speedup vs baseline: 1.1322x; 1.1322x over previous
import functools

import jax
import jax.numpy as jnp
from jax import lax
from jax.experimental import pallas as pl
from jax.experimental.pallas import tpu as pltpu
from jax.experimental.pallas import tpu_sc as plsc

E = 64
D = 768
F = 3072
T = 4096
C = 80
CP = 88
SLOTS = E * CP
TPAD = T + 8
ZERO_SLOT = C

NC, NS = 2, 16
NW = NC * NS

BLK = 512
NB = T // BLK

FB = 768
NFB = F // FB


def _router_body(x_ref, wr_ref, fidx_ref, cidx_ref, gate_ref, aux_ref):
    x = x_ref[...]
    wr = wr_ref[...]
    logits = lax.dot_general(
        x.astype(jnp.bfloat16), wr.astype(jnp.bfloat16),
        (((1,), (0,)), ((), ())),
        preferred_element_type=jnp.float32)
    m = jnp.max(logits, axis=-1, keepdims=True)
    el = jnp.exp(logits - m)
    s = jnp.sum(el, axis=-1, keepdims=True)
    probs = el / s
    maxp = jnp.max(probs, axis=-1, keepdims=True)
    iota_e = lax.broadcasted_iota(jnp.int32, (T, E), 1)
    cand = jnp.where(probs >= maxp, iota_e, E)
    eidx = jnp.min(cand, axis=-1, keepdims=True)
    onehot = (iota_e == eidx).astype(jnp.float32)

    ti = lax.broadcasted_iota(jnp.int32, (BLK, BLK), 0)
    tj = lax.broadcasted_iota(jnp.int32, (BLK, BLK), 1)
    tri = (tj <= ti).astype(jnp.float32)
    carry = jnp.zeros((1, E), jnp.float32)
    for r in range(NB):
        blk = onehot[r * BLK:(r + 1) * BLK, :]
        within = lax.dot_general(
            tri, blk, (((1,), (0,)), ((), ())),
            preferred_element_type=jnp.float32)
        tot = within + carry
        pos = jnp.sum(tot * blk, axis=-1, keepdims=True) - 1.0
        posi = pos.astype(jnp.int32)
        eb = eidx[r * BLK:(r + 1) * BLK, :]
        keep = posi < C
        slot = eb * CP + posi
        fidx_ref[r * BLK:(r + 1) * BLK, :] = jnp.where(keep, slot, C + 1)
        cidx_ref[r * BLK:(r + 1) * BLK, :] = jnp.where(keep, slot, ZERO_SLOT)
        carry = carry + within[BLK - 1:BLK, :]
    gate_ref[...] = maxp
    pmean = jnp.sum(probs, axis=0, keepdims=True) * (1.0 / T)
    aux_ref[...] = (E / T) * jnp.sum(carry * pmean, axis=1, keepdims=True)


@jax.jit
def _router(x, wr):
    return pl.pallas_call(
        _router_body,
        out_shape=[
            jax.ShapeDtypeStruct((T, 1), jnp.int32),
            jax.ShapeDtypeStruct((T, 1), jnp.int32),
            jax.ShapeDtypeStruct((T, 1), jnp.float32),
            jax.ShapeDtypeStruct((1, 1), jnp.float32),
        ],
    )(x, wr)


def _dispatch_body(x_hbm, fidx_hbm, gate_hbm, buf_hbm, gslot_hbm,
                   fidx_v, gate_v, z_v, rows_v, sem):
    wid = lax.axis_index("c") * NS + lax.axis_index("s")
    base = wid * (SLOTS // NW)
    tb = wid * (T // NW)

    z_v[...] = jnp.zeros((16,), jnp.float32)
    pltpu.sync_copy(z_v.at[pl.ds(0, 8)],
                    gslot_hbm.at[pl.ds(base + C, 8)])
    pltpu.sync_copy(z_v.at[pl.ds(8, 8)],
                    gslot_hbm.at[pl.ds(base + CP + C, 8)])

    pltpu.sync_copy(fidx_hbm.at[pl.ds(tb, T // NW)], fidx_v)
    pltpu.sync_copy(gate_hbm.at[pl.ds(tb, T // NW)], gate_v)
    pltpu.sync_copy(x_hbm.at[pl.ds(tb, T // NW)], rows_v)
    pltpu.async_copy(rows_v, buf_hbm.at[fidx_v], sem).wait()
    pltpu.sync_copy(gate_v, gslot_hbm.at[fidx_v])


@jax.jit
def _dispatch(x, fidx, gate):
    return pl.kernel(
        _dispatch_body,
        out_type=[
            jax.ShapeDtypeStruct((SLOTS, D), jnp.float32),
            jax.ShapeDtypeStruct((SLOTS,), jnp.float32),
        ],
        mesh=plsc.VectorSubcoreMesh(
            core_axis_name="c", subcore_axis_name="s",
            num_cores=NC, num_subcores=NS),
        compiler_params=pltpu.CompilerParams(needs_layout_passes=False),
        scratch_types=[
            pltpu.VMEM((T // NW,), jnp.int32),
            pltpu.VMEM((T // NW,), jnp.float32),
            pltpu.VMEM((16,), jnp.float32),
            pltpu.VMEM((T // NW, D), jnp.float32),
            pltpu.SemaphoreType.DMA,
        ],
    )(x, fidx, gate)


def _ffn_body(buf_ref, w1_ref, w2_ref, gs_ref, y_ref):
    f = pl.program_id(1)
    xb = buf_ref[0]
    xb = jnp.where(jnp.abs(xb) < 3e38, xb, 0.0).astype(jnp.bfloat16)
    w1 = w1_ref[0].astype(jnp.bfloat16)
    h = lax.dot_general(xb, w1, (((1,), (0,)), ((), ())),
                        preferred_element_type=jnp.float32)
    h = jnp.maximum(h, 0.0).astype(jnp.bfloat16)
    w2 = w2_ref[0].astype(jnp.bfloat16)
    yp = lax.dot_general(h, w2, (((1,), (0,)), ((), ())),
                         preferred_element_type=jnp.float32)

    @pl.when(f == 0)
    def _():
        y_ref[...] = jnp.zeros_like(y_ref)

    y_ref[0] += yp

    @pl.when(f == NFB - 1)
    def _():
        y_ref[0] = y_ref[0] * gs_ref[0]


@jax.jit
def _ffn(buf, w1, w2, gslot):
    return pl.pallas_call(
        _ffn_body,
        grid=(E, NFB),
        in_specs=[
            pl.BlockSpec((1, CP, D), lambda e, f: (e, 0, 0)),
            pl.BlockSpec((1, D, FB), lambda e, f: (e, 0, f)),
            pl.BlockSpec((1, FB, D), lambda e, f: (e, f, 0)),
            pl.BlockSpec((1, CP, 1), lambda e, f: (e, 0, 0)),
        ],
        out_specs=pl.BlockSpec((1, CP, D), lambda e, f: (e, 0, 0)),
        out_shape=jax.ShapeDtypeStruct((E, CP, D), jnp.float32),
    )(buf, w1, w2, gslot)


def _combine_body(y_hbm, cidx_hbm, out_hbm, cidx_v, rows_v, sem):
    wid = lax.axis_index("c") * NS + lax.axis_index("s")
    base = wid * (T // NW)
    pltpu.sync_copy(cidx_hbm.at[pl.ds(base, T // NW)], cidx_v)
    pltpu.async_copy(y_hbm.at[cidx_v], rows_v, sem).wait()
    pltpu.sync_copy(rows_v, out_hbm.at[pl.ds(base, T // NW)])


@jax.jit
def _combine(y_flat, cidx):
    return pl.kernel(
        _combine_body,
        out_type=jax.ShapeDtypeStruct((T, D), jnp.float32),
        mesh=plsc.VectorSubcoreMesh(
            core_axis_name="c", subcore_axis_name="s",
            num_cores=NC, num_subcores=NS),
        compiler_params=pltpu.CompilerParams(needs_layout_passes=False),
        scratch_types=[
            pltpu.VMEM((T // NW,), jnp.int32),
            pltpu.VMEM((T // NW, D), jnp.float32),
            pltpu.SemaphoreType.DMA,
        ],
    )(y_flat, cidx)


def kernel(x, Wr, W1, W2):
    fidx, cidx, gate, aux = _router(x, Wr)
    buf, gslot = _dispatch(x, fidx.reshape(T), gate.reshape(T))
    y = _ffn(buf.reshape(E, CP, D), W1, W2, gslot.reshape(E, CP, 1))
    out = _combine(y.reshape(SLOTS, D), cidx.reshape(T))
    return out, aux.reshape(())


# speedup vs baseline: 1.2746x; 1.1258x over previous
import functools

import jax
import jax.numpy as jnp
from jax import lax
from jax.experimental import pallas as pl
from jax.experimental.pallas import tpu as pltpu
from jax.experimental.pallas import tpu_sc as plsc

E = 64
D = 768
F = 3072
T = 4096
C = 80
CP = 88
SLOTS = E * CP
TPAD = T + 8
ZERO_SLOT = C

NC, NS = 2, 16
NW = NC * NS

BLK = 512
NB = T // BLK

FB = 1536
NFB = F // FB


def _router_body(x_ref, wr_ref, fidx_ref, cidx_ref, gate_ref, aux_ref):
    x = x_ref[...]
    wr = wr_ref[...]
    logits = lax.dot_general(
        x.astype(jnp.bfloat16), wr.astype(jnp.bfloat16),
        (((1,), (0,)), ((), ())),
        preferred_element_type=jnp.float32)
    m = jnp.max(logits, axis=-1, keepdims=True)
    el = jnp.exp(logits - m)
    s = jnp.sum(el, axis=-1, keepdims=True)
    probs = el / s
    maxp = jnp.max(probs, axis=-1, keepdims=True)
    iota_e = lax.broadcasted_iota(jnp.int32, (T, E), 1)
    cand = jnp.where(probs >= maxp, iota_e, E)
    eidx = jnp.min(cand, axis=-1, keepdims=True)
    onehot = (iota_e == eidx).astype(jnp.float32)

    ti = lax.broadcasted_iota(jnp.int32, (BLK, BLK), 0)
    tj = lax.broadcasted_iota(jnp.int32, (BLK, BLK), 1)
    tri = (tj <= ti).astype(jnp.float32)
    carry = jnp.zeros((1, E), jnp.float32)
    for r in range(NB):
        blk = onehot[r * BLK:(r + 1) * BLK, :]
        within = lax.dot_general(
            tri, blk, (((1,), (0,)), ((), ())),
            preferred_element_type=jnp.float32)
        tot = within + carry
        pos = jnp.sum(tot * blk, axis=-1, keepdims=True) - 1.0
        posi = pos.astype(jnp.int32)
        eb = eidx[r * BLK:(r + 1) * BLK, :]
        keep = posi < C
        slot = eb * CP + posi
        fidx_ref[r * BLK:(r + 1) * BLK, :] = jnp.where(keep, slot, C + 1)
        cidx_ref[r * BLK:(r + 1) * BLK, :] = jnp.where(keep, slot, ZERO_SLOT)
        carry = carry + within[BLK - 1:BLK, :]
    gate_ref[...] = maxp
    pmean = jnp.sum(probs, axis=0, keepdims=True) * (1.0 / T)
    aux_ref[...] = (E / T) * jnp.sum(carry * pmean, axis=1, keepdims=True)


@jax.jit
def _router(x, wr):
    return pl.pallas_call(
        _router_body,
        out_shape=[
            jax.ShapeDtypeStruct((T, 1), jnp.int32),
            jax.ShapeDtypeStruct((T, 1), jnp.int32),
            jax.ShapeDtypeStruct((T, 1), jnp.float32),
            jax.ShapeDtypeStruct((1, 1), jnp.float32),
        ],
    )(x, wr)


def _dispatch_body(x_hbm, fidx_hbm, gate_hbm, buf_hbm, gslot_hbm,
                   fidx_v, gate_v, z_v, rows_v, sem):
    wid = lax.axis_index("c") * NS + lax.axis_index("s")
    base = wid * (SLOTS // NW)
    tb = wid * (T // NW)

    z_v[...] = jnp.zeros((16,), jnp.float32)
    pltpu.sync_copy(z_v.at[pl.ds(0, 8)],
                    gslot_hbm.at[pl.ds(base + C, 8)])
    pltpu.sync_copy(z_v.at[pl.ds(8, 8)],
                    gslot_hbm.at[pl.ds(base + CP + C, 8)])

    pltpu.sync_copy(fidx_hbm.at[pl.ds(tb, T // NW)], fidx_v)
    pltpu.sync_copy(gate_hbm.at[pl.ds(tb, T // NW)], gate_v)
    pltpu.sync_copy(x_hbm.at[pl.ds(tb, T // NW)], rows_v)
    pltpu.async_copy(rows_v, buf_hbm.at[fidx_v], sem).wait()
    pltpu.sync_copy(gate_v, gslot_hbm.at[fidx_v])


@jax.jit
def _dispatch(x, fidx, gate):
    return pl.kernel(
        _dispatch_body,
        out_type=[
            jax.ShapeDtypeStruct((SLOTS, D), jnp.float32),
            jax.ShapeDtypeStruct((SLOTS,), jnp.float32),
        ],
        mesh=plsc.VectorSubcoreMesh(
            core_axis_name="c", subcore_axis_name="s",
            num_cores=NC, num_subcores=NS),
        compiler_params=pltpu.CompilerParams(needs_layout_passes=False),
        scratch_types=[
            pltpu.VMEM((T // NW,), jnp.int32),
            pltpu.VMEM((T // NW,), jnp.float32),
            pltpu.VMEM((16,), jnp.float32),
            pltpu.VMEM((T // NW, D), jnp.float32),
            pltpu.SemaphoreType.DMA,
        ],
    )(x, fidx, gate)


def _ffn_body(buf_ref, w1_ref, w2_ref, gs_ref, y_ref):
    f = pl.program_id(1)
    xb = buf_ref[0]
    xb = jnp.where(jnp.abs(xb) < 3e38, xb, 0.0).astype(jnp.bfloat16)
    w1 = w1_ref[0].astype(jnp.bfloat16)
    h = lax.dot_general(xb, w1, (((1,), (0,)), ((), ())),
                        preferred_element_type=jnp.float32)
    h = jnp.maximum(h, 0.0).astype(jnp.bfloat16)
    w2 = w2_ref[0].astype(jnp.bfloat16)
    yp = lax.dot_general(h, w2, (((1,), (0,)), ((), ())),
                         preferred_element_type=jnp.float32)

    @pl.when(f == 0)
    def _():
        y_ref[...] = jnp.zeros_like(y_ref)

    y_ref[0] += yp

    @pl.when(f == NFB - 1)
    def _():
        y_ref[0] = y_ref[0] * gs_ref[0]


@jax.jit
def _ffn(buf, w1, w2, gslot):
    return pl.pallas_call(
        _ffn_body,
        grid=(E, NFB),
        in_specs=[
            pl.BlockSpec((1, CP, D), lambda e, f: (e, 0, 0)),
            pl.BlockSpec((1, D, FB), lambda e, f: (e, 0, f)),
            pl.BlockSpec((1, FB, D), lambda e, f: (e, f, 0)),
            pl.BlockSpec((1, CP, 1), lambda e, f: (e, 0, 0)),
        ],
        out_specs=pl.BlockSpec((1, CP, D), lambda e, f: (e, 0, 0)),
        out_shape=jax.ShapeDtypeStruct((E, CP, D), jnp.float32),
    )(buf, w1, w2, gslot)


def _combine_body(y_hbm, cidx_hbm, out_hbm, cidx_v, rows_v, sem):
    wid = lax.axis_index("c") * NS + lax.axis_index("s")
    base = wid * (T // NW)
    pltpu.sync_copy(cidx_hbm.at[pl.ds(base, T // NW)], cidx_v)
    pltpu.async_copy(y_hbm.at[cidx_v], rows_v, sem).wait()
    pltpu.sync_copy(rows_v, out_hbm.at[pl.ds(base, T // NW)])


@jax.jit
def _combine(y_flat, cidx):
    return pl.kernel(
        _combine_body,
        out_type=jax.ShapeDtypeStruct((T, D), jnp.float32),
        mesh=plsc.VectorSubcoreMesh(
            core_axis_name="c", subcore_axis_name="s",
            num_cores=NC, num_subcores=NS),
        compiler_params=pltpu.CompilerParams(needs_layout_passes=False),
        scratch_types=[
            pltpu.VMEM((T // NW,), jnp.int32),
            pltpu.VMEM((T // NW, D), jnp.float32),
            pltpu.SemaphoreType.DMA,
        ],
    )(y_flat, cidx)


def kernel(x, Wr, W1, W2):
    fidx, cidx, gate, aux = _router(x, Wr)
    buf, gslot = _dispatch(x, fidx.reshape(T), gate.reshape(T))
    y = _ffn(buf.reshape(E, CP, D), W1, W2, gslot.reshape(E, CP, 1))
    out = _combine(y.reshape(SLOTS, D), cidx.reshape(T))
    return out, aux.reshape(())
